# baseline (speedup 1.0000x reference)
import functools

import jax
import jax.numpy as jnp
from jax import lax
from jax.experimental import pallas as pl
from jax.experimental.pallas import tpu as pltpu
from jax.experimental.pallas import tpu_sc as plsc

N = 10000
E = 320000
D = 128
C = 40
CP = 64
NC, NS = 2, 16
B = 200
NB = 100
NSLOT = 3
EPAD = NS * NB * B
NPAD = 10240
ZR = NPAD // NS
RB = 400
GRID = N // RB


def _sc_agg_body(with_deg, DF, *refs):
    if with_deg:
        (x_h, s3_h, d3_h, zr_h, zd_h, ones_h, pout_h, dout_h,
         acc, sidx, didx, rows, gsems, ssems, dacc, ones_v, dsems) = refs
    else:
        (x_h, s3_h, d3_h, zr_h, pout_h,
         acc, sidx, didx, rows, gsems, ssems) = refs
    c = lax.axis_index("c")
    s = lax.axis_index("s")
    pltpu.sync_copy(zr_h.at[pl.ds(s * ZR, ZR)], acc.at[pl.ds(s * ZR, ZR)])
    if with_deg:
        pltpu.sync_copy(zd_h.at[pl.ds(s * ZR, ZR)], dacc.at[pl.ds(s * ZR, ZR)])
        pltpu.sync_copy(ones_h, ones_v)
    pltpu.sync_copy(s3_h.at[s], sidx)
    pltpu.sync_copy(d3_h.at[s], didx)
    plsc.subcore_barrier()

    def gather(j, u):
        return pltpu.make_async_copy(
            x_h.at[c].at[sidx.at[j]], rows.at[j % NSLOT], gsems.at[u])

    def scat(j, u):
        return pltpu.make_async_copy(
            rows.at[j % NSLOT], acc.at[didx.at[j]], ssems.at[u])

    def dscat(j, u):
        return pltpu.make_async_copy(ones_v, dacc.at[didx.at[j]], dsems.at[u])

    gather(0, 0).start()

    def step(i, carry):
        for u in range(2):
            j = 2 * i + u
            v = (u + 1) % 2
            gather(j, u).wait()
            jp = j - 2

            @pl.when(j >= 2)
            def _():
                scat(jp, u).wait()
                if with_deg:
                    dscat(jp, u).wait()

            pltpu.async_copy(rows.at[j % NSLOT], acc.at[didx.at[j]],
                             ssems.at[u], add=True)
            if with_deg:
                pltpu.async_copy(ones_v, dacc.at[didx.at[j]], dsems.at[u],
                                 add=True)

            @pl.when(j + 1 < NB)
            def _():
                gather(j + 1, v).start()
        return carry

    lax.fori_loop(0, NB // 2, step, 0)
    for jt in (NB - 2, NB - 1):
        scat(jt, jt % 2).wait()
        if with_deg:
            dscat(jt, jt % 2).wait()
    plsc.subcore_barrier()
    pltpu.sync_copy(acc.at[pl.ds(s * ZR, ZR)], pout_h.at[c].at[pl.ds(s * ZR, ZR)])
    if with_deg:
        pltpu.sync_copy(dacc.at[pl.ds(s * ZR, ZR)], dout_h.at[c].at[pl.ds(s * ZR, ZR)])


def _make_sc_agg(with_deg, DF):
    mesh = plsc.VectorSubcoreMesh(core_axis_name="c", subcore_axis_name="s")
    out_type = [jax.ShapeDtypeStruct((NC, NPAD, DF), jnp.float32)]
    scratch = [
        pltpu.VMEM_SHARED((NPAD, DF), jnp.float32),
        pltpu.VMEM((NB, B), jnp.int32),
        pltpu.VMEM((NB, B), jnp.int32),
        pltpu.VMEM((NSLOT, B, DF), jnp.float32),
        pltpu.SemaphoreType.DMA((2,)),
        pltpu.SemaphoreType.DMA((2,)),
    ]
    if with_deg:
        out_type.append(jax.ShapeDtypeStruct((NC, NPAD), jnp.float32))
        scratch += [
            pltpu.VMEM_SHARED((NPAD,), jnp.float32),
            pltpu.VMEM((B,), jnp.float32),
            pltpu.SemaphoreType.DMA((2,)),
        ]
    return pl.kernel(
        functools.partial(_sc_agg_body, with_deg, DF),
        out_type=out_type,
        mesh=mesh,
        scratch_types=scratch,
        compiler_params=pltpu.CompilerParams(use_tc_tiling_on_sc=False),
    )


def _tc1_body(x_ref, p_ref, dg_ref, w1s_ref, w1n_ref, b1_ref,
              w2n_ref, w2s_ref, b2_ref, t_ref, s2_ref):
    psum = jnp.concatenate([p_ref[0], p_ref[1]], axis=1)
    deg = jnp.maximum(dg_ref[0], 1.0)
    n1 = psum / deg
    h = (jnp.dot(x_ref[...], w1s_ref[...], preferred_element_type=jnp.float32)
         + jnp.dot(n1, w1n_ref[...], preferred_element_type=jnp.float32)
         + b1_ref[...])
    h = jnp.maximum(h, 0.0)
    t = jnp.dot(h, w2n_ref[...], preferred_element_type=jnp.float32)
    t_ref[0] = t[:, :CP // 2]
    t_ref[1] = t[:, CP // 2:]
    s2_ref[...] = (jnp.dot(h, w2s_ref[...], preferred_element_type=jnp.float32)
                   + b2_ref[...])


def _tc2_body(s2_ref, q_ref, dg_ref, o_ref):
    deg = jnp.maximum(dg_ref[0], 1.0)
    qsum = jnp.concatenate([q_ref[0], q_ref[1]], axis=1)
    z = s2_ref[...][:, :C] + (qsum / deg)[:, :C]
    m = jnp.max(z, axis=-1, keepdims=True)
    zz = z - m
    lse = jnp.log(jnp.sum(jnp.exp(zz), axis=-1, keepdims=True))
    o_ref[...] = zz - lse


_sc_agg_deg = _make_sc_agg(True, D // 2)
_sc_agg_plain = _make_sc_agg(False, CP // 2)

_tc1 = pl.pallas_call(
    _tc1_body,
    grid=(GRID,),
    in_specs=[
        pl.BlockSpec((RB, D), lambda i: (i, 0)),
        pl.BlockSpec((NC, RB, D // 2), lambda i: (0, i, 0)),
        pl.BlockSpec((NC, RB, 1), lambda i: (0, i, 0)),
        pl.BlockSpec((D, D), lambda i: (0, 0)),
        pl.BlockSpec((D, D), lambda i: (0, 0)),
        pl.BlockSpec((1, D), lambda i: (0, 0)),
        pl.BlockSpec((D, CP), lambda i: (0, 0)),
        pl.BlockSpec((D, CP), lambda i: (0, 0)),
        pl.BlockSpec((1, CP), lambda i: (0, 0)),
    ],
    out_specs=[
        pl.BlockSpec((NC, RB, CP // 2), lambda i: (0, i, 0)),
        pl.BlockSpec((RB, CP), lambda i: (i, 0)),
    ],
    out_shape=[
        jax.ShapeDtypeStruct((NC, N, CP // 2), jnp.float32),
        jax.ShapeDtypeStruct((N, CP), jnp.float32),
    ],
)

_tc2 = pl.pallas_call(
    _tc2_body,
    grid=(GRID,),
    in_specs=[
        pl.BlockSpec((RB, CP), lambda i: (i, 0)),
        pl.BlockSpec((NC, RB, CP // 2), lambda i: (0, i, 0)),
        pl.BlockSpec((NC, RB, 1), lambda i: (0, i, 0)),
    ],
    out_specs=pl.BlockSpec((RB, C), lambda i: (i, 0)),
    out_shape=jax.ShapeDtypeStruct((N, C), jnp.float32),
)


def kernel(x, edge_index, W1_self, W1_neigh, b1, W2_self, W2_neigh, b2):
    src = edge_index[0]
    dst = edge_index[1]
    src_p = src.reshape(NS, NB, B)
    dst_p = dst.reshape(NS, NB, B)
    zr = jnp.zeros((NPAD, D // 2), jnp.float32)
    zr2 = jnp.zeros((NPAD, CP // 2), jnp.float32)
    zd = jnp.zeros((NPAD,), jnp.float32)
    ones_b = jnp.ones((B,), jnp.float32)
    x_split = jnp.stack([x[:, :D // 2], x[:, D // 2:]])

    p1, dgp = _sc_agg_deg(x_split, src_p, dst_p, zr, zd, ones_b)
    dgp3 = dgp.reshape(NC, NPAD, 1)

    w2n_p = jnp.pad(W2_neigh, ((0, 0), (0, CP - C)))
    w2s_p = jnp.pad(W2_self, ((0, 0), (0, CP - C)))
    b2_p = jnp.pad(b2, (0, CP - C)).reshape(1, CP)
    t, s2 = _tc1(x, p1, dgp3, W1_self, W1_neigh, b1.reshape(1, D),
                  w2n_p, w2s_p, b2_p)

    (q,) = _sc_agg_plain(t, src_p, dst_p, zr2)
    return _tc2(s2, q, dgp3)


# speedup vs baseline: 1.0834x; 1.0834x over previous
import functools

import jax
import jax.numpy as jnp
from jax import lax
from jax.experimental import pallas as pl
from jax.experimental.pallas import tpu as pltpu
from jax.experimental.pallas import tpu_sc as plsc

N = 10000
E = 320000
D = 128
C = 40
CP = 64
NC, NS = 2, 16
B = 200
NB = 100
EPAD = NS * NB * B
NPAD = 10240
ZR = NPAD // NS
RB = 400
GRID = N // RB


def _sc_agg_body(with_deg, split_cols, NBk, Bk, SL, *refs):
    if with_deg:
        (x_h, s3_h, d3_h, zr_h, zd_h, ones_h, pout_h, dout_h,
         acc, sidx, didx, rows, gsems, ssems, dacc, ones_v, dsems) = refs
    else:
        (x_h, s3_h, d3_h, zr_h, pout_h,
         acc, sidx, didx, rows, gsems, ssems) = refs
    c = lax.axis_index("c")
    s = lax.axis_index("s")
    w = s if split_cols else s * NC + c
    NB, B = NBk, Bk
    pltpu.sync_copy(zr_h.at[pl.ds(s * ZR, ZR)], acc.at[pl.ds(s * ZR, ZR)])
    if with_deg:
        pltpu.sync_copy(zd_h.at[pl.ds(s * ZR, ZR)], dacc.at[pl.ds(s * ZR, ZR)])
        pltpu.sync_copy(ones_h, ones_v)
    pltpu.sync_copy(s3_h.at[w], sidx)
    pltpu.sync_copy(d3_h.at[w], didx)
    plsc.subcore_barrier()

    def gather(j, u):
        gsrc = x_h.at[c] if split_cols else x_h
        return pltpu.make_async_copy(
            gsrc.at[sidx.at[j]], rows.at[j % SL], gsems.at[u])

    def scat(j, u):
        return pltpu.make_async_copy(
            rows.at[j % SL], acc.at[didx.at[j]], ssems.at[u])

    def dscat(j, u):
        return pltpu.make_async_copy(ones_v, dacc.at[didx.at[j]], dsems.at[u])

    gather(0, 0).start()

    def step(i, carry):
        for u in range(2):
            j = 2 * i + u
            v = (u + 1) % 2
            gather(j, u).wait()
            jp = j - (SL - 1)
            up = (u - (SL - 1)) % 2

            @pl.when(j >= SL - 1)
            def _():
                scat(jp, up).wait()
                if with_deg:
                    dscat(jp, up).wait()

            pltpu.async_copy(rows.at[j % SL], acc.at[didx.at[j]],
                             ssems.at[u], add=True)
            if with_deg:
                pltpu.async_copy(ones_v, dacc.at[didx.at[j]], dsems.at[u],
                                 add=True)

            @pl.when(j + 1 < NB)
            def _():
                gather(j + 1, v).start()
        return carry

    lax.fori_loop(0, NB // 2, step, 0)
    for jt in range(NB - SL + 1, NB):
        scat(jt, jt % 2).wait()
        if with_deg:
            dscat(jt, jt % 2).wait()
    plsc.subcore_barrier()
    pltpu.sync_copy(acc.at[pl.ds(s * ZR, ZR)], pout_h.at[c].at[pl.ds(s * ZR, ZR)])
    if with_deg:
        pltpu.sync_copy(dacc.at[pl.ds(s * ZR, ZR)], dout_h.at[c].at[pl.ds(s * ZR, ZR)])


def _make_sc_agg(with_deg, split_cols, DF, NBk, Bk, SL):
    mesh = plsc.VectorSubcoreMesh(core_axis_name="c", subcore_axis_name="s")
    out_type = [jax.ShapeDtypeStruct((NC, NPAD, DF), jnp.float32)]
    scratch = [
        pltpu.VMEM_SHARED((NPAD, DF), jnp.float32),
        pltpu.VMEM((NBk, Bk), jnp.int32),
        pltpu.VMEM((NBk, Bk), jnp.int32),
        pltpu.VMEM((SL, Bk, DF), jnp.float32),
        pltpu.SemaphoreType.DMA((2,)),
        pltpu.SemaphoreType.DMA((2,)),
    ]
    if with_deg:
        out_type.append(jax.ShapeDtypeStruct((NC, NPAD), jnp.float32))
        scratch += [
            pltpu.VMEM_SHARED((NPAD,), jnp.float32),
            pltpu.VMEM((Bk,), jnp.float32),
            pltpu.SemaphoreType.DMA((2,)),
        ]
    return pl.kernel(
        functools.partial(_sc_agg_body, with_deg, split_cols, NBk, Bk, SL),
        out_type=out_type,
        mesh=mesh,
        scratch_types=scratch,
        compiler_params=pltpu.CompilerParams(use_tc_tiling_on_sc=False),
    )


def _tc1_body(x_ref, p_ref, dg_ref, w1s_ref, w1n_ref, b1_ref,
              w2n_ref, w2s_ref, b2_ref, t_ref, s2_ref):
    psum = jnp.concatenate([p_ref[0], p_ref[1]], axis=1)
    deg = jnp.maximum(dg_ref[0], 1.0)
    n1 = psum / deg
    h = (jnp.dot(x_ref[...], w1s_ref[...], preferred_element_type=jnp.float32)
         + jnp.dot(n1, w1n_ref[...], preferred_element_type=jnp.float32)
         + b1_ref[...])
    h = jnp.maximum(h, 0.0)
    t = jnp.dot(h, w2n_ref[...], preferred_element_type=jnp.float32)
    t_ref[...] = jnp.concatenate(
        [t, jnp.zeros((RB, CP - C), jnp.float32)], axis=1)
    s2_ref[...] = (jnp.dot(h, w2s_ref[...], preferred_element_type=jnp.float32)
                   + b2_ref[...])


def _tc2_body(s2_ref, q_ref, dg_ref, o_ref):
    deg = jnp.maximum(dg_ref[0], 1.0)
    qsum = q_ref[0] + q_ref[1]
    z = s2_ref[...] + (qsum / deg)[:, :C]
    m = jnp.max(z, axis=-1, keepdims=True)
    zz = z - m
    lse = jnp.log(jnp.sum(jnp.exp(zz), axis=-1, keepdims=True))
    o_ref[...] = zz - lse


NB2, B2 = 50, 200
_sc_agg_deg = _make_sc_agg(True, True, D // 2, NB, B, 3)
_sc_agg_plain = _make_sc_agg(False, False, CP, NB2, B2, 2)

_tc1 = pl.pallas_call(
    _tc1_body,
    grid=(GRID,),
    in_specs=[
        pl.BlockSpec((RB, D), lambda i: (i, 0)),
        pl.BlockSpec((NC, RB, D // 2), lambda i: (0, i, 0)),
        pl.BlockSpec((NC, RB, 1), lambda i: (0, i, 0)),
        pl.BlockSpec((D, D), lambda i: (0, 0)),
        pl.BlockSpec((D, D), lambda i: (0, 0)),
        pl.BlockSpec((1, D), lambda i: (0, 0)),
        pl.BlockSpec((D, C), lambda i: (0, 0)),
        pl.BlockSpec((D, C), lambda i: (0, 0)),
        pl.BlockSpec((1, C), lambda i: (0, 0)),
    ],
    out_specs=[
        pl.BlockSpec((RB, CP), lambda i: (i, 0)),
        pl.BlockSpec((RB, C), lambda i: (i, 0)),
    ],
    out_shape=[
        jax.ShapeDtypeStruct((N, CP), jnp.float32),
        jax.ShapeDtypeStruct((N, C), jnp.float32),
    ],
)

_tc2 = pl.pallas_call(
    _tc2_body,
    grid=(GRID,),
    in_specs=[
        pl.BlockSpec((RB, C), lambda i: (i, 0)),
        pl.BlockSpec((NC, RB, CP), lambda i: (0, i, 0)),
        pl.BlockSpec((NC, RB, 1), lambda i: (0, i, 0)),
    ],
    out_specs=pl.BlockSpec((RB, C), lambda i: (i, 0)),
    out_shape=jax.ShapeDtypeStruct((N, C), jnp.float32),
)


def kernel(x, edge_index, W1_self, W1_neigh, b1, W2_self, W2_neigh, b2):
    src = edge_index[0]
    dst = edge_index[1]
    src_p = src.reshape(NS, NB, B)
    dst_p = dst.reshape(NS, NB, B)
    src_p2 = src.reshape(NC * NS, NB2, B2)
    dst_p2 = dst.reshape(NC * NS, NB2, B2)
    zr = jnp.zeros((NPAD, D // 2), jnp.float32)
    zr2 = jnp.zeros((NPAD, CP), jnp.float32)
    zd = jnp.zeros((NPAD,), jnp.float32)
    ones_b = jnp.ones((B,), jnp.float32)
    x_split = jnp.stack([x[:, :D // 2], x[:, D // 2:]])

    p1, dgp = _sc_agg_deg(x_split, src_p, dst_p, zr, zd, ones_b)
    dgp3 = dgp.reshape(NC, NPAD, 1)

    t, s2 = _tc1(x, p1, dgp3, W1_self, W1_neigh, b1.reshape(1, D),
                 W2_neigh, W2_self, b2.reshape(1, C))

    (q,) = _sc_agg_plain(t, src_p2, dst_p2, zr2)
    return _tc2(s2, q, dgp3)
